# baseline (speedup 1.0000x reference)
import functools
import math

import jax
import jax.numpy as jnp
from jax import lax
from jax.experimental import pallas as pl
from jax.experimental.pallas import tpu as pltpu
from jax.experimental.pallas import tpu_sc as plsc

EMBED_W = 64
SCALE = math.sqrt(64.0)
CHUNK = 128
GRP = 8


def _make_lookup(n_rows: int):
    info = plsc.get_sparse_core_info()
    nc, ns = info.num_cores, info.num_subcores
    nw = nc * ns
    rows_per_w = n_rows // nw
    n_steps = rows_per_w // GRP
    assert n_steps * GRP == rows_per_w

    mesh = plsc.VectorSubcoreMesh(core_axis_name="c", subcore_axis_name="s")

    @functools.partial(
        pl.kernel,
        mesh=mesh,
        out_type=jax.ShapeDtypeStruct((n_rows, CHUNK, EMBED_W), jnp.float32),
        scratch_types=[
            pltpu.VMEM((GRP, CHUNK), jnp.int32),
            pltpu.VMEM((GRP, CHUNK, EMBED_W), jnp.float32),
            pltpu.SemaphoreType.DMA,
        ],
        compiler_params=pltpu.CompilerParams(use_tc_tiling_on_sc=False),
    )
    def lookup(lut_hbm, idx_hbm, out_hbm, idx_v, rows_v, sem):
        wid = lax.axis_index("s") * nc + lax.axis_index("c")
        base = wid * rows_per_w

        def step(g, carry):
            rb = base + g * GRP
            pltpu.sync_copy(idx_hbm.at[pl.ds(rb, GRP)], idx_v)
            copies = [
                pltpu.async_copy(lut_hbm.at[idx_v.at[j]], rows_v.at[j], sem)
                for j in range(GRP)
            ]
            for c in copies:
                c.wait()

            def scale_row(r, inner):
                for j in range(GRP):
                    for s in range(EMBED_W // 16):
                        sl = pl.ds(s * 16, 16)
                        rows_v[j, r, sl] = rows_v[j, r, sl] * SCALE
                return inner

            lax.fori_loop(0, CHUNK, scale_row, 0)
            pltpu.sync_copy(rows_v, out_hbm.at[pl.ds(rb, GRP)])
            return carry

        lax.fori_loop(0, n_steps, step, 0)

    return lookup


def kernel(x, lut):
    b, s = x.shape
    idx = x.reshape(-1, CHUNK).astype(jnp.int32)
    out = _make_lookup(idx.shape[0])(lut, idx)
    return out.reshape(b, s, EMBED_W)


# speedup vs baseline: 1.0644x; 1.0644x over previous
import functools
import math

import jax
import jax.numpy as jnp
from jax import lax
from jax.experimental import pallas as pl
from jax.experimental.pallas import tpu as pltpu
from jax.experimental.pallas import tpu_sc as plsc

EMBED_W = 64
SCALE = math.sqrt(64.0)
CHUNK = 128
GRP = 4
NBUF = 2


def _make_lookup(n_rows: int):
    info = plsc.get_sparse_core_info()
    nc, ns = info.num_cores, info.num_subcores
    nw = nc * ns
    rows_per_w = n_rows // nw
    n_outer = rows_per_w // (GRP * NBUF)
    assert n_outer * GRP * NBUF == rows_per_w

    mesh = plsc.VectorSubcoreMesh(core_axis_name="c", subcore_axis_name="s")

    @functools.partial(
        pl.kernel,
        mesh=mesh,
        out_type=jax.ShapeDtypeStruct((n_rows, CHUNK, EMBED_W), jnp.float32),
        scratch_types=[
            pltpu.VMEM((rows_per_w, CHUNK), jnp.int32),
            pltpu.VMEM((NBUF, GRP, CHUNK, EMBED_W), jnp.float32),
            pltpu.SemaphoreType.DMA,
            pltpu.SemaphoreType.DMA,
            pltpu.SemaphoreType.DMA,
            pltpu.SemaphoreType.DMA,
        ],
        compiler_params=pltpu.CompilerParams(use_tc_tiling_on_sc=False),
    )
    def lookup(lut_hbm, idx_hbm, out_hbm, idx_v, rows_v, g0, g1, w0, w1):
        wid = lax.axis_index("s") * nc + lax.axis_index("c")
        base = wid * rows_per_w
        g_sems = [g0, g1]
        w_sems = [w0, w1]

        pltpu.sync_copy(idx_hbm.at[pl.ds(base, rows_per_w)], idx_v)

        def outer(t, carry):
            copies = []
            for b in range(NBUF):
                @pl.when(t > 0)
                def _drain(b=b):
                    pltpu.make_async_copy(
                        rows_v.at[b], out_hbm.at[pl.ds(base, GRP)], w_sems[b]
                    ).wait()

                g = t * NBUF + b
                copies.append([
                    pltpu.async_copy(
                        lut_hbm.at[idx_v.at[g * GRP + j]],
                        rows_v.at[b, j],
                        g_sems[b],
                    )
                    for j in range(GRP)
                ])

            for b in range(NBUF):
                for c in copies[b]:
                    c.wait()

                def scale_row(r, inner, b=b):
                    for j in range(GRP):
                        for s in range(EMBED_W // 16):
                            sl = pl.ds(s * 16, 16)
                            rows_v[b, j, r, sl] = rows_v[b, j, r, sl] * SCALE
                    return inner

                lax.fori_loop(0, CHUNK, scale_row, 0)
                g = t * NBUF + b
                pltpu.async_copy(
                    rows_v.at[b],
                    out_hbm.at[pl.ds(base + g * GRP, GRP)],
                    w_sems[b],
                )
            return carry

        lax.fori_loop(0, n_outer, outer, 0)

        for b in range(NBUF):
            pltpu.make_async_copy(
                rows_v.at[b], out_hbm.at[pl.ds(base, GRP)], w_sems[b]
            ).wait()

    return lookup


def kernel(x, lut):
    b, s = x.shape
    idx = x.reshape(-1, CHUNK).astype(jnp.int32)
    out = _make_lookup(idx.shape[0])(lut, idx)
    return out.reshape(b, s, EMBED_W)
